# baseline (speedup 1.0000x reference)
import jax
import jax.numpy as jnp
from jax.experimental import pallas as pl

_B, _T, _C_S1, _C_MASK, _H, _W = 32, 2, 3, 2, 256, 256
_G, _OUT_CH = 8, 16
_HW = _H * _W
_THRESH = 0.02 * (_H * _W)


def _body(s1_ref, lai_ref, mask_ref, glob_ref, wct_ref, wgt_ref, out_ref):
    s1 = s1_ref[0]
    lai = lai_ref[0]
    m = mask_ref[0]
    wct = wct_ref[...]
    wgt = wgt_ref[...]
    gcol = glob_ref[0]

    m0_t0 = m[0:1]
    m0_t1 = m[2:3]
    clouds0 = m0_t0 == 0.0
    clouds1 = m0_t1 == 0.0
    fill0 = jnp.logical_and(clouds0, jnp.logical_not(clouds1))
    fill1 = jnp.logical_and(clouds1, jnp.logical_not(clouds0))
    count0 = jnp.sum(fill0.astype(jnp.float32))
    count1 = jnp.sum(fill1.astype(jnp.float32))
    sel0 = count0 > _THRESH
    sel1 = count1 > _THRESH

    lai0 = jnp.where(jnp.logical_and(sel0, fill0), lai[1:2], lai[0:1])
    lai1 = jnp.where(jnp.logical_and(sel1, fill1), lai[0:1], lai[1:2])
    mask0 = jnp.where(jnp.logical_and(sel0, clouds0), m[2:4], m[0:2])
    mask1 = jnp.where(jnp.logical_and(sel1, clouds1), m[0:2], m[2:4])

    def dg(a, b):
        return jax.lax.dot_general(
            a, b, (((1,), (0,)), ((), ())),
            preferred_element_type=jnp.float32)

    acc = dg(wct[:, 0:3], s1[0:3])
    acc += dg(wct[:, 3:4], lai0)
    acc += dg(wct[:, 4:6], mask0)
    acc += dg(wct[:, 6:9], s1[3:6])
    acc += dg(wct[:, 9:10], lai1)
    acc += dg(wct[:, 10:12], mask1)
    bias = dg(wgt, gcol)
    out_ref[0] = acc + bias


def kernel(s1_data, in_lai, in_mask_lai, glob, W_conv, W_glob):
    s1f = s1_data.reshape(_B, _T * _C_S1, _HW)
    laif = in_lai.reshape(_B, _T, _HW)
    maskf = in_mask_lai.reshape(_B, _T * _C_MASK, _HW)
    globt = glob.reshape(_B, _G, 1)
    wct = W_conv.T
    wgt = W_glob.T

    out = pl.pallas_call(
        _body,
        grid=(_B,),
        in_specs=[
            pl.BlockSpec((1, _T * _C_S1, _HW), lambda b: (b, 0, 0)),
            pl.BlockSpec((1, _T, _HW), lambda b: (b, 0, 0)),
            pl.BlockSpec((1, _T * _C_MASK, _HW), lambda b: (b, 0, 0)),
            pl.BlockSpec((1, _G, 1), lambda b: (b, 0, 0)),
            pl.BlockSpec((_OUT_CH, _T * (_C_S1 + 1 + _C_MASK)), lambda b: (0, 0)),
            pl.BlockSpec((_OUT_CH, _G), lambda b: (0, 0)),
        ],
        out_specs=pl.BlockSpec((1, _OUT_CH, _HW), lambda b: (b, 0, 0)),
        out_shape=jax.ShapeDtypeStruct((_B, _OUT_CH, _HW), jnp.float32),
    )(s1f, laif, maskf, globt, wct, wgt)
    return out.reshape(_B, _OUT_CH, _H, _W)


# speedup vs baseline: 1.9313x; 1.9313x over previous
import jax
import jax.numpy as jnp
from jax.experimental import pallas as pl
from jax.experimental.pallas import tpu as pltpu

_B, _T, _C_S1, _C_MASK, _H, _W = 32, 2, 3, 2, 256, 256
_G, _OUT_CH = 8, 16
_C_IN = _T * (_C_S1 + 1 + _C_MASK)
_THRESH = 0.02 * (_H * _W)


def _body(s1_ref, lai_ref, mask_ref, glob_ref, wc_ref, wg_ref, out_ref):
    glob_row = glob_ref
    m00 = mask_ref[0, 0, 0]
    m10 = mask_ref[0, 1, 0]
    m01 = mask_ref[0, 0, 1]
    m11 = mask_ref[0, 1, 1]
    clouds0 = m00 == 0.0
    clouds1 = m10 == 0.0
    fill0 = jnp.logical_and(clouds0, jnp.logical_not(clouds1))
    fill1 = jnp.logical_and(clouds1, jnp.logical_not(clouds0))
    sel0 = jnp.sum(fill0.astype(jnp.float32)) > _THRESH
    sel1 = jnp.sum(fill1.astype(jnp.float32)) > _THRESH

    lai0 = jnp.where(jnp.logical_and(sel0, fill0), lai_ref[0, 1, 0], lai_ref[0, 0, 0])
    lai1 = jnp.where(jnp.logical_and(sel1, fill1), lai_ref[0, 0, 0], lai_ref[0, 1, 0])
    rep0 = jnp.logical_and(sel0, clouds0)
    rep1 = jnp.logical_and(sel1, clouds1)
    m00f = jnp.where(rep0, m10, m00)
    m01f = jnp.where(rep0, m11, m01)
    m10f = jnp.where(rep1, m00, m10)
    m11f = jnp.where(rep1, m01, m11)

    feats = [
        s1_ref[0, 0, 0], s1_ref[0, 0, 1], s1_ref[0, 0, 2], lai0, m00f, m01f,
        s1_ref[0, 1, 0], s1_ref[0, 1, 1], s1_ref[0, 1, 2], lai1, m10f, m11f,
    ]
    for o in range(_OUT_CH):
        bias = glob_row[0, 0, 0] * wg_ref[0, o]
        for g in range(1, _G):
            bias += glob_row[0, 0, g] * wg_ref[g, o]
        acc = feats[0] * wc_ref[0, o]
        for c in range(1, _C_IN):
            acc += feats[c] * wc_ref[c, o]
        out_ref[0, o] = acc + bias


def kernel(s1_data, in_lai, in_mask_lai, glob, W_conv, W_glob):
    return pl.pallas_call(
        _body,
        grid=(_B,),
        in_specs=[
            pl.BlockSpec((1, _T, _C_S1, _H, _W), lambda b: (b, 0, 0, 0, 0)),
            pl.BlockSpec((1, _T, 1, _H, _W), lambda b: (b, 0, 0, 0, 0)),
            pl.BlockSpec((1, _T, _C_MASK, _H, _W), lambda b: (b, 0, 0, 0, 0)),
            pl.BlockSpec((1, 1, _G), lambda b: (b, 0, 0), memory_space=pltpu.SMEM),
            pl.BlockSpec((_C_IN, _OUT_CH), lambda b: (0, 0), memory_space=pltpu.SMEM),
            pl.BlockSpec((_G, _OUT_CH), lambda b: (0, 0), memory_space=pltpu.SMEM),
        ],
        out_specs=pl.BlockSpec((1, _OUT_CH, _H, _W), lambda b: (b, 0, 0, 0)),
        out_shape=jax.ShapeDtypeStruct((_B, _OUT_CH, _H, _W), jnp.float32),
    )(s1_data, in_lai, in_mask_lai, glob.reshape(_B, 1, _G), W_conv, W_glob)


# speedup vs baseline: 3.0235x; 1.5655x over previous
import jax
import jax.numpy as jnp
from jax.experimental import pallas as pl
from jax.experimental.pallas import tpu as pltpu

_B, _T, _C_S1, _C_MASK, _H, _W = 32, 2, 3, 2, 256, 256
_G, _OUT_CH = 8, 16
_C_IN = _T * (_C_S1 + 1 + _C_MASK)
_THRESH = 0.02 * (_H * _W)
_SH = 16


def _body(s1_ref, lai_ref, mask_ref, glob_ref, wc_ref, wg_ref, out_ref):
    clouds0 = mask_ref[0, 0, 0] == 0.0
    clouds1 = mask_ref[0, 1, 0] == 0.0
    fill0 = jnp.logical_and(clouds0, jnp.logical_not(clouds1))
    fill1 = jnp.logical_and(clouds1, jnp.logical_not(clouds0))
    sel0 = jnp.sum(fill0.astype(jnp.float32)) > _THRESH
    sel1 = jnp.sum(fill1.astype(jnp.float32)) > _THRESH

    biases = []
    for o in range(_OUT_CH):
        b = glob_ref[0, 0, 0] * wg_ref[0, o]
        for g in range(1, _G):
            b += glob_ref[0, 0, g] * wg_ref[g, o]
        biases.append(b)

    for s in range(_H // _SH):
        sl = pl.ds(s * _SH, _SH)
        m00 = mask_ref[0, 0, 0, sl, :]
        m01 = mask_ref[0, 0, 1, sl, :]
        m10 = mask_ref[0, 1, 0, sl, :]
        m11 = mask_ref[0, 1, 1, sl, :]
        c0 = m00 == 0.0
        c1 = m10 == 0.0
        f0 = jnp.logical_and(sel0, jnp.logical_and(c0, jnp.logical_not(c1)))
        f1 = jnp.logical_and(sel1, jnp.logical_and(c1, jnp.logical_not(c0)))
        r0 = jnp.logical_and(sel0, c0)
        r1 = jnp.logical_and(sel1, c1)
        la0 = lai_ref[0, 0, 0, sl, :]
        la1 = lai_ref[0, 1, 0, sl, :]
        feats = [
            s1_ref[0, 0, 0, sl, :], s1_ref[0, 0, 1, sl, :], s1_ref[0, 0, 2, sl, :],
            jnp.where(f0, la1, la0),
            jnp.where(r0, m10, m00), jnp.where(r0, m11, m01),
            s1_ref[0, 1, 0, sl, :], s1_ref[0, 1, 1, sl, :], s1_ref[0, 1, 2, sl, :],
            jnp.where(f1, la0, la1),
            jnp.where(r1, m00, m10), jnp.where(r1, m01, m11),
        ]
        for o in range(_OUT_CH):
            acc = feats[0] * wc_ref[0, o] + biases[o]
            for c in range(1, _C_IN):
                acc += feats[c] * wc_ref[c, o]
            out_ref[0, o, sl, :] = acc


def kernel(s1_data, in_lai, in_mask_lai, glob, W_conv, W_glob):
    return pl.pallas_call(
        _body,
        grid=(_B,),
        in_specs=[
            pl.BlockSpec((1, _T, _C_S1, _H, _W), lambda b: (b, 0, 0, 0, 0)),
            pl.BlockSpec((1, _T, 1, _H, _W), lambda b: (b, 0, 0, 0, 0)),
            pl.BlockSpec((1, _T, _C_MASK, _H, _W), lambda b: (b, 0, 0, 0, 0)),
            pl.BlockSpec((1, 1, _G), lambda b: (b, 0, 0), memory_space=pltpu.SMEM),
            pl.BlockSpec((_C_IN, _OUT_CH), lambda b: (0, 0), memory_space=pltpu.SMEM),
            pl.BlockSpec((_G, _OUT_CH), lambda b: (0, 0), memory_space=pltpu.SMEM),
        ],
        out_specs=pl.BlockSpec((1, _OUT_CH, _H, _W), lambda b: (b, 0, 0, 0)),
        out_shape=jax.ShapeDtypeStruct((_B, _OUT_CH, _H, _W), jnp.float32),
    )(s1_data, in_lai, in_mask_lai, glob.reshape(_B, 1, _G), W_conv, W_glob)


# speedup vs baseline: 3.2747x; 1.0831x over previous
import jax
import jax.numpy as jnp
from jax.experimental import pallas as pl
from jax.experimental.pallas import tpu as pltpu

_B, _T, _C_S1, _C_MASK, _H, _W = 32, 2, 3, 2, 256, 256
_G, _OUT_CH = 8, 16
_C_IN = _T * (_C_S1 + 1 + _C_MASK)
_THRESH = 0.02 * (_H * _W)
_SH = 16


def _body(s1_ref, lai_ref, mask_ref, glob_ref, wc_ref, wg_ref, out_ref):
    clouds0 = mask_ref[0, 0, 0] == 0.0
    clouds1 = mask_ref[0, 1, 0] == 0.0
    fill0 = jnp.logical_and(clouds0, jnp.logical_not(clouds1))
    fill1 = jnp.logical_and(clouds1, jnp.logical_not(clouds0))
    sel0 = jnp.sum(fill0.astype(jnp.float32)) > _THRESH
    sel1 = jnp.sum(fill1.astype(jnp.float32)) > _THRESH

    biases = []
    for o in range(_OUT_CH):
        b = glob_ref[0, 0, 0] * wg_ref[0, o]
        for g in range(1, _G):
            b += glob_ref[0, 0, g] * wg_ref[g, o]
        biases.append(b)
    wc = [[wc_ref[c, o] for o in range(_OUT_CH)] for c in range(_C_IN)]

    for s in range(_H // _SH):
        sl = pl.ds(s * _SH, _SH)
        m00 = mask_ref[0, 0, 0, sl, :]
        m01 = mask_ref[0, 0, 1, sl, :]
        m10 = mask_ref[0, 1, 0, sl, :]
        m11 = mask_ref[0, 1, 1, sl, :]
        c0 = m00 == 0.0
        c1 = m10 == 0.0
        f0 = jnp.logical_and(sel0, jnp.logical_and(c0, jnp.logical_not(c1)))
        f1 = jnp.logical_and(sel1, jnp.logical_and(c1, jnp.logical_not(c0)))
        r0 = jnp.logical_and(sel0, c0)
        r1 = jnp.logical_and(sel1, c1)
        la0 = lai_ref[0, 0, 0, sl, :]
        la1 = lai_ref[0, 1, 0, sl, :]
        feats = [
            s1_ref[0, 0, 0, sl, :], s1_ref[0, 0, 1, sl, :], s1_ref[0, 0, 2, sl, :],
            jnp.where(f0, la1, la0),
            jnp.where(r0, m10, m00), jnp.where(r0, m11, m01),
            s1_ref[0, 1, 0, sl, :], s1_ref[0, 1, 1, sl, :], s1_ref[0, 1, 2, sl, :],
            jnp.where(f1, la0, la1),
            jnp.where(r1, m00, m10), jnp.where(r1, m01, m11),
        ]
        for o in range(_OUT_CH):
            acc = feats[0] * wc[0][o] + biases[o]
            for c in range(1, _C_IN):
                acc += feats[c] * wc[c][o]
            out_ref[0, o, sl, :] = acc


def kernel(s1_data, in_lai, in_mask_lai, glob, W_conv, W_glob):
    return pl.pallas_call(
        _body,
        grid=(_B,),
        in_specs=[
            pl.BlockSpec((1, _T, _C_S1, _H, _W), lambda b: (b, 0, 0, 0, 0)),
            pl.BlockSpec((1, _T, 1, _H, _W), lambda b: (b, 0, 0, 0, 0)),
            pl.BlockSpec((1, _T, _C_MASK, _H, _W), lambda b: (b, 0, 0, 0, 0)),
            pl.BlockSpec((1, 1, _G), lambda b: (b, 0, 0), memory_space=pltpu.SMEM),
            pl.BlockSpec((_C_IN, _OUT_CH), lambda b: (0, 0), memory_space=pltpu.SMEM),
            pl.BlockSpec((_G, _OUT_CH), lambda b: (0, 0), memory_space=pltpu.SMEM),
        ],
        out_specs=pl.BlockSpec((1, _OUT_CH, _H, _W), lambda b: (b, 0, 0, 0)),
        out_shape=jax.ShapeDtypeStruct((_B, _OUT_CH, _H, _W), jnp.float32),
        compiler_params=pltpu.CompilerParams(
            dimension_semantics=("parallel",)),
    )(s1_data, in_lai, in_mask_lai, glob.reshape(_B, 1, _G), W_conv, W_glob)


# speedup vs baseline: 4.8637x; 1.4852x over previous
import jax
import jax.numpy as jnp
from jax.experimental import pallas as pl
from jax.experimental.pallas import tpu as pltpu

_B, _T, _C_S1, _C_MASK, _H, _W = 32, 2, 3, 2, 256, 256
_G, _OUT_CH = 8, 16
_C_IN = _T * (_C_S1 + 1 + _C_MASK)
_THRESH = 0.02 * (_H * _W)
_SH = 16


def _body(s1_ref, lai_ref, mask_ref, glob_ref, a_ref, wg_ref, out_ref):
    clouds0 = mask_ref[0, 0, 0] == 0.0
    clouds1 = mask_ref[0, 1, 0] == 0.0
    fill0 = jnp.logical_and(clouds0, jnp.logical_not(clouds1))
    fill1 = jnp.logical_and(clouds1, jnp.logical_not(clouds0))
    sel0 = jnp.sum(fill0.astype(jnp.float32)) > _THRESH
    sel1 = jnp.sum(fill1.astype(jnp.float32)) > _THRESH

    biases = []
    for o in range(_OUT_CH):
        b = glob_ref[0, 0, 0] * wg_ref[0, o]
        for g in range(1, _G):
            b += glob_ref[0, 0, g] * wg_ref[g, o]
        biases.append(b)

    a_mat = a_ref[...]

    for s in range(_H // _SH):
        sl = pl.ds(s * _SH, _SH)
        m00 = mask_ref[0, 0, 0, sl, :]
        m01 = mask_ref[0, 0, 1, sl, :]
        m10 = mask_ref[0, 1, 0, sl, :]
        m11 = mask_ref[0, 1, 1, sl, :]
        c0 = m00 == 0.0
        c1 = m10 == 0.0
        f0 = jnp.logical_and(sel0, jnp.logical_and(c0, jnp.logical_not(c1)))
        f1 = jnp.logical_and(sel1, jnp.logical_and(c1, jnp.logical_not(c0)))
        r0 = jnp.logical_and(sel0, c0)
        r1 = jnp.logical_and(sel1, c1)
        la0 = lai_ref[0, 0, 0, sl, :]
        la1 = lai_ref[0, 1, 0, sl, :]
        x = jnp.concatenate([
            s1_ref[0, 0, 0, sl, :], s1_ref[0, 0, 1, sl, :], s1_ref[0, 0, 2, sl, :],
            jnp.where(f0, la1, la0),
            jnp.where(r0, m10, m00), jnp.where(r0, m11, m01),
            s1_ref[0, 1, 0, sl, :], s1_ref[0, 1, 1, sl, :], s1_ref[0, 1, 2, sl, :],
            jnp.where(f1, la0, la1),
            jnp.where(r1, m00, m10), jnp.where(r1, m01, m11),
        ], axis=0)
        out_strip = jax.lax.dot_general(
            a_mat, x, (((1,), (0,)), ((), ())),
            preferred_element_type=jnp.float32)
        for o in range(_OUT_CH):
            out_ref[0, o, sl, :] = out_strip[o * _SH:(o + 1) * _SH, :] + biases[o]


def kernel(s1_data, in_lai, in_mask_lai, glob, W_conv, W_glob):
    a_mat = jnp.kron(W_conv.T, jnp.eye(_SH, dtype=jnp.float32))
    return pl.pallas_call(
        _body,
        grid=(_B,),
        in_specs=[
            pl.BlockSpec((1, _T, _C_S1, _H, _W), lambda b: (b, 0, 0, 0, 0)),
            pl.BlockSpec((1, _T, 1, _H, _W), lambda b: (b, 0, 0, 0, 0)),
            pl.BlockSpec((1, _T, _C_MASK, _H, _W), lambda b: (b, 0, 0, 0, 0)),
            pl.BlockSpec((1, 1, _G), lambda b: (b, 0, 0), memory_space=pltpu.SMEM),
            pl.BlockSpec((_OUT_CH * _SH, _C_IN * _SH), lambda b: (0, 0)),
            pl.BlockSpec((_G, _OUT_CH), lambda b: (0, 0), memory_space=pltpu.SMEM),
        ],
        out_specs=pl.BlockSpec((1, _OUT_CH, _H, _W), lambda b: (b, 0, 0, 0)),
        out_shape=jax.ShapeDtypeStruct((_B, _OUT_CH, _H, _W), jnp.float32),
        compiler_params=pltpu.CompilerParams(
            dimension_semantics=("parallel",)),
    )(s1_data, in_lai, in_mask_lai, glob.reshape(_B, 1, _G), a_mat, W_glob)
